# baseline (speedup 1.0000x reference)
import functools

import jax
import jax.numpy as jnp
import numpy as np
from jax import lax
from jax.experimental import pallas as pl
from jax.experimental.pallas import tpu as pltpu
from jax.experimental.pallas import tpu_sc as plsc

_NE = 8
_NC = 1000
_EC = 16
_GC = 8
_B = 128
_HW = 224
_OHW = 112
_NPIX = _OHW * _OHW
_M = _GC + _NE * _EC
_ME = _NE * _EC

_RB = 8
_KR = 24
_K = 9 * _KR
_NBLK = _OHW // _RB
_G = 4
_PW = 32


def _sel_matrix():
    e = np.zeros((_HW, 256), dtype=np.float32)
    for dj in range(2):
        for j in range(_OHW):
            e[2 * j + dj, 128 * dj + j] = 1.0
    return jnp.asarray(e)


def _row_onehot():
    m = np.zeros((3, _RB, _KR), dtype=np.float32)
    for di in range(3):
        for t in range(_RB):
            m[di, t, 2 * t + di] = 1.0
    return jnp.asarray(m)


def _banded(w):
    a5 = jnp.einsum('abcd,cef->eadbf', w, _row_onehot())
    return a5.reshape(_RB * w.shape[0], _K)


def _make_bb(p3, blk):
    rows = p3[:, 16 * blk: 16 * blk + _KR, :]
    g0 = rows[:, :, 0:128]
    g1 = rows[:, :, 128:256]
    g2 = jnp.pad(g0[:, :, 1:], ((0, 0), (0, 0), (0, 1)))
    return jnp.stack([g0, g1, g2], axis=0).reshape(_K, 128)


def _convpool_body(x_ref, e_ref, ag_ref, ae_ref, gwl_ref, gbl_ref, out_ref):
    for g in range(_G):
        xb = x_ref[g]
        xr = xb.reshape(3 * _HW, _HW)
        p3 = jnp.dot(xr, e_ref[:], preferred_element_type=jnp.float32)
        p3 = p3.reshape(3, _HW, 256)
        p3 = jnp.pad(p3, ((0, 0), (0, 16), (0, 0)))

        acc_g = jnp.zeros((_RB * _GC, 128), dtype=jnp.float32)
        for blk in range(_NBLK):
            conv = jnp.dot(ag_ref[:], _make_bb(p3, blk),
                           preferred_element_type=jnp.float32)
            acc_g = acc_g + jnp.maximum(conv, 0.0)
        pooled_g = jnp.sum(acc_g.reshape(_RB, _GC, 128), axis=(0, 2)) \
            * (1.0 / _NPIX)

        lg = jnp.dot(pooled_g[None, :], gwl_ref[:],
                     preferred_element_type=jnp.float32) + gbl_ref[:]
        mx = jnp.max(lg)
        iota8 = jax.lax.broadcasted_iota(jnp.int32, (1, _NE), 1)
        idx = jnp.min(jnp.where(lg == mx, iota8, _NE))

        ae_sel = ae_ref[pl.ds(idx * _ME, _ME), :]
        acc_e = jnp.zeros((_RB * _EC, 128), dtype=jnp.float32)
        for blk in range(_NBLK):
            conv = jnp.dot(ae_sel, _make_bb(p3, blk),
                           preferred_element_type=jnp.float32)
            acc_e = acc_e + jnp.maximum(conv, 0.0)
        pooled_e = jnp.sum(acc_e.reshape(_RB, _EC, 128), axis=(0, 2)) \
            * (1.0 / _NPIX)

        out_ref[g, 0] = jnp.concatenate(
            [pooled_g, pooled_e, jnp.full((8,), idx, jnp.float32)])


def _sc_route_body(packedT_hbm, gwx_hbm, zt_hbm, probsT_hbm, ohwT_hbm,
                   pk_v, gwx_v, zt_v, probsT_v, ohw_v):
    wid = lax.axis_index("s") * 2 + lax.axis_index("c")

    @pl.when(wid < 8)
    def _():
        pltpu.sync_copy(packedT_hbm.at[wid], pk_v)
        pltpu.sync_copy(gwx_hbm, gwx_v)
        pg = [pk_v[g, :] for g in range(_GC)]
        idxf = pk_v[_GC + _EC, :]
        lg = []
        for e_i in range(_NE):
            acc = gwx_v[64 + e_i, :]
            for g in range(_GC):
                acc = acc + gwx_v[g * _NE + e_i, :] * pg[g]
            lg.append(acc)
        mx = lg[0]
        for e_i in range(1, _NE):
            mx = jnp.maximum(mx, lg[e_i])
        ex = [jnp.exp(l - mx) for l in lg]
        tot = ex[0]
        for e_i in range(1, _NE):
            tot = tot + ex[e_i]
        inv = 1.0 / tot
        probs = [ex[e_i] * inv for e_i in range(_NE)]
        bw = jnp.zeros((16,), jnp.float32)
        for e_i in range(_NE):
            probsT_v[e_i, :] = probs[e_i]
            bw = bw + jnp.where(idxf == float(e_i), probs[e_i], 0.0)
        ohw = [jnp.where(idxf == float(e_i), bw, 0.0)
               for e_i in range(_NE)]
        for e_i in range(_NE):
            ohw_v[e_i, :] = ohw[e_i]
        for k in range(_EC):
            pe_k = pk_v[_GC + k, :]
            for e_i in range(_NE):
                zt_v[e_i * _EC + k, :] = pe_k * ohw[e_i]
        pltpu.sync_copy(zt_v, zt_hbm.at[wid])
        pltpu.sync_copy(probsT_v, probsT_hbm.at[wid])
        pltpu.sync_copy(ohw_v, ohwT_hbm.at[wid])


def _combine_body(zt_ref, ohwT_ref, probsT_ref, wl_ref, bl_ref,
                  out_ref, aux_ref):
    out_ref[:] = (
        lax.dot_general(zt_ref[:], wl_ref[:], (((0,), (0,)), ((), ())),
                        preferred_element_type=jnp.float32)
        + lax.dot_general(ohwT_ref[:], bl_ref[:], (((0,), (0,)), ((), ())),
                          preferred_element_type=jnp.float32))
    mean_probs = jnp.mean(probsT_ref[:], axis=1)
    aux_ref[0, 0] = jnp.mean((mean_probs - 1.0 / _NE) ** 2)


@jax.jit
def kernel(x, gate_wc, gate_wl, gate_bl, exp_wc, exp_wl, exp_bl):
    ag = _banded(gate_wc.reshape(_GC, 3, 3, 3))
    ae = _banded(exp_wc.reshape(_ME, 3, 3, 3))
    ae = ae.reshape(_RB, _NE, _EC, _K).transpose(1, 0, 2, 3).reshape(
        _NE * _RB * _EC, _K)
    e_mat = _sel_matrix()

    packed = pl.pallas_call(
        _convpool_body,
        grid=(_B // _G,),
        in_specs=[
            pl.BlockSpec((_G, 3, _HW, _HW), lambda b: (b, 0, 0, 0)),
            pl.BlockSpec((_HW, 256), lambda b: (0, 0)),
            pl.BlockSpec((_RB * _GC, _K), lambda b: (0, 0)),
            pl.BlockSpec((_NE * _RB * _EC, _K), lambda b: (0, 0)),
            pl.BlockSpec((_NE, _NE), lambda b: (0, 0)),
            pl.BlockSpec((1, _NE), lambda b: (0, 0)),
        ],
        out_specs=pl.BlockSpec((_G, 1, _PW), lambda b: (b, 0, 0)),
        out_shape=jax.ShapeDtypeStruct((_B, 1, _PW), jnp.float32),
    )(x, e_mat, ag, ae, gate_wl, gate_bl.reshape(1, _NE))
    packed3 = packed.reshape(8, 16, _PW).transpose(0, 2, 1)
    gwx = jnp.broadcast_to(
        jnp.concatenate([gate_wl.reshape(64), gate_bl])[:, None],
        (72, 16))

    sc_route = functools.partial(
        pl.kernel,
        mesh=plsc.VectorSubcoreMesh(core_axis_name="c", subcore_axis_name="s"),
        out_type=[
            jax.ShapeDtypeStruct((8, _ME, 16), jnp.float32),
            jax.ShapeDtypeStruct((8, _NE, 16), jnp.float32),
            jax.ShapeDtypeStruct((8, _NE, 16), jnp.float32),
        ],
        scratch_types=[
            pltpu.VMEM((_PW, 16), jnp.float32),
            pltpu.VMEM((72, 16), jnp.float32),
            pltpu.VMEM((_ME, 16), jnp.float32),
            pltpu.VMEM((_NE, 16), jnp.float32),
            pltpu.VMEM((_NE, 16), jnp.float32),
        ],
    )(_sc_route_body)
    zt3, probs_t3, ohw_t3 = sc_route(packed3, gwx)
    zt = zt3.transpose(1, 0, 2).reshape(_ME, _B)
    probs_t = probs_t3.transpose(1, 0, 2).reshape(_NE, _B)
    ohw_t = ohw_t3.transpose(1, 0, 2).reshape(_NE, _B)

    out, aux = pl.pallas_call(
        _combine_body,
        in_specs=[pl.BlockSpec(memory_space=pltpu.VMEM)] * 5,
        out_specs=[
            pl.BlockSpec(memory_space=pltpu.VMEM),
            pl.BlockSpec(memory_space=pltpu.SMEM),
        ],
        out_shape=[
            jax.ShapeDtypeStruct((_B, _NC), jnp.float32),
            jax.ShapeDtypeStruct((1, 1), jnp.float32),
        ],
    )(zt, ohw_t, probs_t, exp_wl.reshape(_NE * _EC, _NC), exp_bl)

    return out, probs_t.T, aux.reshape(())


# speedup vs baseline: 1.0208x; 1.0208x over previous
import functools

import jax
import jax.numpy as jnp
import numpy as np
from jax import lax
from jax.experimental import pallas as pl
from jax.experimental.pallas import tpu as pltpu
from jax.experimental.pallas import tpu_sc as plsc

_NE = 8
_NC = 1000
_EC = 16
_GC = 8
_B = 128
_HW = 224
_OHW = 112
_NPIX = _OHW * _OHW
_M = _GC + _NE * _EC
_ME = _NE * _EC

_RB = 8
_KR = 24
_K = 9 * _KR
_NBLK = _OHW // _RB
_G = 8
_PW = 32


def _sel_matrix():
    e = np.zeros((_HW, 256), dtype=np.float32)
    for dj in range(2):
        for j in range(_OHW):
            e[2 * j + dj, 128 * dj + j] = 1.0
    return jnp.asarray(e)


def _row_onehot():
    m = np.zeros((3, _RB, _KR), dtype=np.float32)
    for di in range(3):
        for t in range(_RB):
            m[di, t, 2 * t + di] = 1.0
    return jnp.asarray(m)


def _banded(w):
    a5 = jnp.einsum('abcd,cef->eadbf', w, _row_onehot())
    return a5.reshape(_RB * w.shape[0], _K)


def _make_bb(p3, blk):
    rows = p3[:, 16 * blk: 16 * blk + _KR, :]
    g0 = rows[:, :, 0:128]
    g1 = rows[:, :, 128:256]
    g2 = jnp.pad(g0[:, :, 1:], ((0, 0), (0, 0), (0, 1)))
    return jnp.stack([g0, g1, g2], axis=0).reshape(_K, 128)


def _convpool_body(x_ref, e_ref, ag_ref, ae_ref, gwl_ref, gbl_ref, out_ref):
    for g in range(_G):
        xb = x_ref[g]
        xr = xb.reshape(3 * _HW, _HW)
        p3 = jnp.dot(xr, e_ref[:], preferred_element_type=jnp.float32)
        p3 = p3.reshape(3, _HW, 256)
        p3 = jnp.pad(p3, ((0, 0), (0, 16), (0, 0)))

        acc_g = jnp.zeros((_RB * _GC, 128), dtype=jnp.float32)
        for blk in range(_NBLK):
            conv = jnp.dot(ag_ref[:], _make_bb(p3, blk),
                           preferred_element_type=jnp.float32)
            acc_g = acc_g + jnp.maximum(conv, 0.0)
        pooled_g = jnp.sum(acc_g.reshape(_RB, _GC, 128), axis=(0, 2)) \
            * (1.0 / _NPIX)

        lg = jnp.dot(pooled_g[None, :], gwl_ref[:],
                     preferred_element_type=jnp.float32) + gbl_ref[:]
        mx = jnp.max(lg)
        iota8 = jax.lax.broadcasted_iota(jnp.int32, (1, _NE), 1)
        idx = jnp.min(jnp.where(lg == mx, iota8, _NE))

        ae_sel = ae_ref[pl.ds(idx * _ME, _ME), :]
        acc_e = jnp.zeros((_RB * _EC, 128), dtype=jnp.float32)
        for blk in range(_NBLK):
            conv = jnp.dot(ae_sel, _make_bb(p3, blk),
                           preferred_element_type=jnp.float32)
            acc_e = acc_e + jnp.maximum(conv, 0.0)
        pooled_e = jnp.sum(acc_e.reshape(_RB, _EC, 128), axis=(0, 2)) \
            * (1.0 / _NPIX)

        out_ref[g, 0] = jnp.concatenate(
            [pooled_g, pooled_e, jnp.full((8,), idx, jnp.float32)])


def _sc_route_body(packedT_hbm, gwx_hbm, zt_hbm, probsT_hbm, ohwT_hbm,
                   pk_v, gwx_v, zt_v, probsT_v, ohw_v):
    wid = lax.axis_index("s") * 2 + lax.axis_index("c")

    @pl.when(wid == 0)
    def _():
        pltpu.sync_copy(packedT_hbm, pk_v)
        pltpu.sync_copy(gwx_hbm, gwx_v)
        for chunk in range(8):
            sl = pl.ds(chunk * 16, 16)
            pg = [pk_v[g, sl] for g in range(_GC)]
            idxf = pk_v[_GC + _EC, sl]
            lg = []
            for e_i in range(_NE):
                acc = gwx_v[64 + e_i, :]
                for g in range(_GC):
                    acc = acc + gwx_v[g * _NE + e_i, :] * pg[g]
                lg.append(acc)
            mx = lg[0]
            for e_i in range(1, _NE):
                mx = jnp.maximum(mx, lg[e_i])
            ex = [jnp.exp(l - mx) for l in lg]
            tot = ex[0]
            for e_i in range(1, _NE):
                tot = tot + ex[e_i]
            inv = 1.0 / tot
            probs = [ex[e_i] * inv for e_i in range(_NE)]
            bw = jnp.zeros((16,), jnp.float32)
            for e_i in range(_NE):
                probsT_v[e_i, sl] = probs[e_i]
                bw = bw + jnp.where(idxf == float(e_i), probs[e_i], 0.0)
            ohw = [jnp.where(idxf == float(e_i), bw, 0.0)
                   for e_i in range(_NE)]
            for e_i in range(_NE):
                ohw_v[e_i, sl] = ohw[e_i]
            for k in range(_EC):
                pe_k = pk_v[_GC + k, sl]
                for e_i in range(_NE):
                    zt_v[e_i * _EC + k, sl] = pe_k * ohw[e_i]
        pltpu.sync_copy(zt_v, zt_hbm)
        pltpu.sync_copy(probsT_v, probsT_hbm)
        pltpu.sync_copy(ohw_v, ohwT_hbm)


def _combine_body(zt_ref, ohwT_ref, probsT_ref, wl_ref, bl_ref,
                  out_ref, aux_ref):
    out_ref[:] = (
        lax.dot_general(zt_ref[:], wl_ref[:], (((0,), (0,)), ((), ())),
                        preferred_element_type=jnp.float32)
        + lax.dot_general(ohwT_ref[:], bl_ref[:], (((0,), (0,)), ((), ())),
                          preferred_element_type=jnp.float32))
    mean_probs = jnp.mean(probsT_ref[:], axis=1)
    aux_ref[0, 0] = jnp.mean((mean_probs - 1.0 / _NE) ** 2)


@jax.jit
def kernel(x, gate_wc, gate_wl, gate_bl, exp_wc, exp_wl, exp_bl):
    ag = _banded(gate_wc.reshape(_GC, 3, 3, 3))
    ae = _banded(exp_wc.reshape(_ME, 3, 3, 3))
    ae = ae.reshape(_RB, _NE, _EC, _K).transpose(1, 0, 2, 3).reshape(
        _NE * _RB * _EC, _K)
    e_mat = _sel_matrix()

    packed = pl.pallas_call(
        _convpool_body,
        grid=(_B // _G,),
        in_specs=[
            pl.BlockSpec((_G, 3, _HW, _HW), lambda b: (b, 0, 0, 0)),
            pl.BlockSpec((_HW, 256), lambda b: (0, 0)),
            pl.BlockSpec((_RB * _GC, _K), lambda b: (0, 0)),
            pl.BlockSpec((_NE * _RB * _EC, _K), lambda b: (0, 0)),
            pl.BlockSpec((_NE, _NE), lambda b: (0, 0)),
            pl.BlockSpec((1, _NE), lambda b: (0, 0)),
        ],
        out_specs=pl.BlockSpec((_G, 1, _PW), lambda b: (b, 0, 0)),
        out_shape=jax.ShapeDtypeStruct((_B, 1, _PW), jnp.float32),
    )(x, e_mat, ag, ae, gate_wl, gate_bl.reshape(1, _NE))
    packedT = packed.reshape(_B, _PW).T
    gwx = jnp.broadcast_to(
        jnp.concatenate([gate_wl.reshape(64), gate_bl])[:, None],
        (72, 16))

    sc_route = functools.partial(
        pl.kernel,
        mesh=plsc.VectorSubcoreMesh(core_axis_name="c", subcore_axis_name="s"),
        out_type=[
            jax.ShapeDtypeStruct((_ME, _B), jnp.float32),
            jax.ShapeDtypeStruct((_NE, _B), jnp.float32),
            jax.ShapeDtypeStruct((_NE, _B), jnp.float32),
        ],
        scratch_types=[
            pltpu.VMEM((_PW, _B), jnp.float32),
            pltpu.VMEM((72, 16), jnp.float32),
            pltpu.VMEM((_ME, _B), jnp.float32),
            pltpu.VMEM((_NE, _B), jnp.float32),
            pltpu.VMEM((_NE, _B), jnp.float32),
        ],
    )(_sc_route_body)
    zt, probs_t, ohw_t = sc_route(packedT, gwx)

    out, aux = pl.pallas_call(
        _combine_body,
        in_specs=[pl.BlockSpec(memory_space=pltpu.VMEM)] * 5,
        out_specs=[
            pl.BlockSpec(memory_space=pltpu.VMEM),
            pl.BlockSpec(memory_space=pltpu.SMEM),
        ],
        out_shape=[
            jax.ShapeDtypeStruct((_B, _NC), jnp.float32),
            jax.ShapeDtypeStruct((1, 1), jnp.float32),
        ],
    )(zt, ohw_t, probs_t, exp_wl.reshape(_NE * _EC, _NC), exp_bl)

    return out, probs_t.T, aux.reshape(())
